# baseline (speedup 1.0000x reference)
import functools

import jax
import jax.numpy as jnp
from jax import lax
from jax.experimental import pallas as pl
from jax.experimental.pallas import tpu as pltpu
from jax.experimental.pallas import tpu_sc as plsc


def _embed_kernel(F, V, D, B):
    NC, NS = 2, 16
    NW = NC * NS
    IDX_PER_GATHER = 128
    B_CHUNK = 64
    ROWS_CHUNK = B_CHUNK * F
    N_GATHER = ROWS_CHUNK // IDX_PER_GATHER
    assert ROWS_CHUNK % IDX_PER_GATHER == 0
    B_PER_W = B // NW
    N_CHUNK = B_PER_W // B_CHUNK
    assert B % (NW * B_CHUNK) == 0

    mesh = plsc.VectorSubcoreMesh(core_axis_name="c", subcore_axis_name="s")

    @functools.partial(
        pl.kernel,
        out_type=jax.ShapeDtypeStruct((B * F, D), jnp.float32),
        mesh=mesh,
        scratch_types=[
            pltpu.VMEM((ROWS_CHUNK,), jnp.int32),
            pltpu.VMEM((ROWS_CHUNK, D), jnp.float32),
            pltpu.SemaphoreType.DMA,
        ],
        compiler_params=pltpu.CompilerParams(use_tc_tiling_on_sc=False),
    )
    def k(idx_hbm, tab_hbm, out_hbm, idx_v, rows_v, sem):
        wid = lax.axis_index("s") * NC + lax.axis_index("c")
        row0 = wid * (B_PER_W * F)

        def chunk_body(c, carry):
            base = row0 + c * ROWS_CHUNK
            pltpu.sync_copy(idx_hbm.at[pl.ds(base, ROWS_CHUNK)], idx_v)
            copies = []
            for j in range(N_GATHER):
                copies.append(pltpu.async_copy(
                    tab_hbm.at[idx_v.at[pl.ds(j * IDX_PER_GATHER,
                                              IDX_PER_GATHER)]],
                    rows_v.at[pl.ds(j * IDX_PER_GATHER, IDX_PER_GATHER)],
                    sem))
            for cp in copies:
                cp.wait()
            pltpu.sync_copy(rows_v, out_hbm.at[pl.ds(base, ROWS_CHUNK)])
            return carry

        lax.fori_loop(0, N_CHUNK, chunk_body, 0)

    return k


def kernel(indices, tables):
    F, B = indices.shape
    _, V, D = tables.shape
    tab_flat = tables.reshape(F * V, D)
    offs = (jnp.arange(F, dtype=jnp.int32) * V)[:, None]
    idx_flat = (indices.astype(jnp.int32) + offs).T.reshape(-1)
    out = _embed_kernel(F, V, D, B)(idx_flat, tab_flat)
    return out.reshape(B, F * D)


# speedup vs baseline: 1.0010x; 1.0010x over previous
import functools

import jax
import jax.numpy as jnp
from jax import lax
from jax.experimental import pallas as pl
from jax.experimental.pallas import tpu as pltpu
from jax.experimental.pallas import tpu_sc as plsc


def _embed_kernel(F, V, D, B):
    NC, NS = 2, 16
    NW = NC * NS
    IDX_PER_GATHER = 128
    B_CHUNK = 64
    ROWS_CHUNK = B_CHUNK * F
    N_GATHER = ROWS_CHUNK // IDX_PER_GATHER
    assert ROWS_CHUNK % IDX_PER_GATHER == 0
    B_PER_W = B // NW
    N_CHUNK = B_PER_W // B_CHUNK
    assert B % (NW * B_CHUNK) == 0

    mesh = plsc.VectorSubcoreMesh(core_axis_name="c", subcore_axis_name="s")

    ROWS_W = B_PER_W * F

    @functools.partial(
        pl.kernel,
        out_type=jax.ShapeDtypeStruct((B * F, D), jnp.float32),
        mesh=mesh,
        scratch_types=[
            pltpu.VMEM((ROWS_W,), jnp.int32),
            pltpu.VMEM((ROWS_CHUNK, D), jnp.float32),
            pltpu.VMEM((ROWS_CHUNK, D), jnp.float32),
            pltpu.SemaphoreType.DMA,
            pltpu.SemaphoreType.DMA,
            pltpu.SemaphoreType.DMA,
            pltpu.SemaphoreType.DMA,
        ],
        compiler_params=pltpu.CompilerParams(use_tc_tiling_on_sc=False),
    )
    def k(idx_hbm, tab_hbm, out_hbm, idx_v, rows0, rows1, sg0, sg1, sw0, sw1):
        wid = lax.axis_index("s") * NC + lax.axis_index("c")
        row0 = wid * ROWS_W
        rows = (rows0, rows1)
        sg = (sg0, sg1)
        sw = (sw0, sw1)

        pltpu.sync_copy(idx_hbm.at[pl.ds(row0, ROWS_W)], idx_v)

        def fire_g(c, r):
            for j in range(N_GATHER):
                pltpu.async_copy(
                    tab_hbm.at[idx_v.at[pl.ds(c * ROWS_CHUNK + j * IDX_PER_GATHER,
                                              IDX_PER_GATHER)]],
                    rows[r].at[pl.ds(j * IDX_PER_GATHER, IDX_PER_GATHER)],
                    sg[r])

        def wait_g(r):
            pltpu.make_async_copy(
                tab_hbm.at[pl.ds(0, ROWS_CHUNK)], rows[r], sg[r]).wait()

        def fire_w(c, r):
            pltpu.async_copy(
                rows[r], out_hbm.at[pl.ds(row0 + c * ROWS_CHUNK, ROWS_CHUNK)],
                sw[r])

        def wait_w(r):
            pltpu.make_async_copy(
                rows[r], out_hbm.at[pl.ds(0, ROWS_CHUNK)], sw[r]).wait()

        fire_g(0, 0)

        def body(i, carry):
            c = 2 * i
            wait_g(0)
            fire_w(c, 0)

            @pl.when(i >= 1)
            def _():
                wait_w(1)
            fire_g(c + 1, 1)

            wait_g(1)
            fire_w(c + 1, 1)

            @pl.when(i < N_CHUNK // 2 - 1)
            def _():
                wait_w(0)
                fire_g(c + 2, 0)
            return carry

        lax.fori_loop(0, N_CHUNK // 2, body, 0)
        wait_w(0)
        wait_w(1)

    return k


def kernel(indices, tables):
    F, B = indices.shape
    _, V, D = tables.shape
    tab_flat = tables.reshape(F * V, D)
    offs = (jnp.arange(F, dtype=jnp.int32) * V)[:, None]
    idx_flat = (indices.astype(jnp.int32) + offs).T.reshape(-1)
    out = _embed_kernel(F, V, D, B)(idx_flat, tab_flat)
    return out.reshape(B, F * D)


# speedup vs baseline: 1.0049x; 1.0039x over previous
import functools

import jax
import jax.numpy as jnp
from jax import lax
from jax.experimental import pallas as pl
from jax.experimental.pallas import tpu as pltpu
from jax.experimental.pallas import tpu_sc as plsc


def _embed_kernel(F, V, D, B):
    NC, NS = 2, 16
    NW = NC * NS
    B_PER_W = B // NW
    B_CHUNK = 64
    N_CHUNK = B_PER_W // B_CHUNK
    assert B % (NW * B_CHUNK) == 0 and N_CHUNK % 2 == 0

    mesh = plsc.VectorSubcoreMesh(core_axis_name="c", subcore_axis_name="s")

    @functools.partial(
        pl.kernel,
        out_type=jax.ShapeDtypeStruct((B, F * D), jnp.float32),
        mesh=mesh,
        scratch_types=[
            pltpu.VMEM((F, B_PER_W), jnp.int32),
            pltpu.VMEM((F, B_CHUNK, D), jnp.float32),
            pltpu.VMEM((F, B_CHUNK, D), jnp.float32),
            pltpu.SemaphoreType.DMA,
            pltpu.SemaphoreType.DMA,
            pltpu.SemaphoreType.DMA,
            pltpu.SemaphoreType.DMA,
        ],
        compiler_params=pltpu.CompilerParams(use_tc_tiling_on_sc=False),
    )
    def k(idx_hbm, tab_hbm, out_hbm, idx_v, rows0, rows1, sg0, sg1, sw0, sw1):
        wid = lax.axis_index("s") * NC + lax.axis_index("c")
        b0 = wid * B_PER_W
        rows = (rows0, rows1)
        sg = (sg0, sg1)
        sw = (sw0, sw1)

        pltpu.sync_copy(idx_hbm.at[:, pl.ds(b0, B_PER_W)], idx_v)

        def fire_g(c, r):
            for f in range(F):
                pltpu.async_copy(
                    tab_hbm.at[f].at[idx_v.at[f, pl.ds(c * B_CHUNK, B_CHUNK)]],
                    rows[r].at[f],
                    sg[r])

        def wait_g(r):
            pltpu.make_async_copy(
                tab_hbm.at[:, pl.ds(0, B_CHUNK), :], rows[r], sg[r]).wait()

        def fire_w(c, r):
            for f in range(F):
                pltpu.async_copy(
                    rows[r].at[f],
                    out_hbm.at[pl.ds(b0 + c * B_CHUNK, B_CHUNK),
                               pl.ds(f * D, D)],
                    sw[r])

        def wait_w(r):
            pltpu.make_async_copy(
                tab_hbm.at[:, pl.ds(0, B_CHUNK), :], rows[r], sw[r]).wait()

        fire_g(0, 0)

        def body(i, carry):
            c = 2 * i
            wait_g(0)
            fire_w(c, 0)

            @pl.when(i >= 1)
            def _():
                wait_w(1)
            fire_g(c + 1, 1)

            wait_g(1)
            fire_w(c + 1, 1)

            @pl.when(i < N_CHUNK // 2 - 1)
            def _():
                wait_w(0)
                fire_g(c + 2, 0)
            return carry

        lax.fori_loop(0, N_CHUNK // 2, body, 0)
        wait_w(0)
        wait_w(1)

    return k


def kernel(indices, tables):
    F, B = indices.shape
    _, V, D = tables.shape
    return _embed_kernel(F, V, D, B)(indices.astype(jnp.int32), tables)
